# baseline (speedup 1.0000x reference)
import functools

import jax
import jax.numpy as jnp
from jax import lax
from jax.experimental import pallas as pl
from jax.experimental.pallas import tpu as pltpu
from jax.experimental.pallas import tpu_sc as plsc

B, C, N, R = 8, 2048, 8192, 5
CHUNK_N = 1024

NQ = 4
NW = 32
QLEN = B * N // NW
CAND = 2 * R * 16


def _proj_body(x_ref, w_ref, out_ref):
    x = x_ref[0]
    w = w_ref[...]
    out_ref[0] = jnp.sum(x * w, axis=0, keepdims=True)


def _project(in_features, conv_w):
    w_col = conv_w.reshape(C, 1)
    return pl.pallas_call(
        _proj_body,
        grid=(B, N // CHUNK_N),
        in_specs=[
            pl.BlockSpec((1, C, CHUNK_N), lambda b, n: (b, 0, n)),
            pl.BlockSpec((C, 1), lambda b, n: (0, 0)),
        ],
        out_specs=pl.BlockSpec((1, 1, CHUNK_N), lambda b, n: (b, 0, n)),
        out_shape=jax.ShapeDtypeStruct((B, 1, N), jnp.float32),
        compiler_params=pltpu.CompilerParams(
            dimension_semantics=("parallel", "parallel"),
        ),
    )(in_features, w_col)


def _select_body(agg_ref, out_ref, row_v, res_v):
    wid = lax.axis_index("s") * 2 + lax.axis_index("c")
    pltpu.sync_copy(agg_ref.at[wid], row_v)
    neg = jnp.full((16,), -jnp.inf, jnp.float32)
    pos = jnp.full((16,), jnp.inf, jnp.float32)

    def body(i, carry):
        ts, us = list(carry[:R]), list(carry[R:])
        v = row_v[pl.ds(i * 16, 16)]
        w = v
        for k in range(R):
            hi = jnp.maximum(ts[k], w)
            w = jnp.minimum(ts[k], w)
            ts[k] = hi
        w = v
        for k in range(R):
            lo = jnp.minimum(us[k], w)
            w = jnp.maximum(us[k], w)
            us[k] = lo
        return tuple(ts) + tuple(us)

    carry = lax.fori_loop(0, QLEN // 16, body, (neg,) * R + (pos,) * R)
    for k in range(2 * R):
        res_v[pl.ds(16 * k, 16)] = carry[k]
    pltpu.sync_copy(res_v, out_ref.at[wid])


def _select(agg32):
    mesh = plsc.VectorSubcoreMesh(core_axis_name="c", subcore_axis_name="s")
    f = functools.partial(
        pl.kernel,
        out_type=jax.ShapeDtypeStruct((NW, CAND), jnp.float32),
        mesh=mesh,
        scratch_types=[
            pltpu.VMEM((QLEN,), jnp.float32),
            pltpu.VMEM((CAND,), jnp.float32),
        ],
    )(_select_body)
    return f(agg32)


def _head_body(cand_ref, b0_ref, w1_ref, b1_ref, w2_ref, b2_ref, wo_ref,
               bo_ref, out_ref):
    a = cand_ref[...]
    w = a.shape[1]
    idx = lax.broadcasted_iota(jnp.int32, (B, w), 1)
    kind = (idx % CAND) // 16
    lane = lax.broadcasted_iota(jnp.int32, (B, 16), 1)
    mil = jnp.zeros((B, 16), jnp.float32)

    work = jnp.where(kind < R, a, -jnp.inf)
    for r in range(R):
        m = jnp.max(work, axis=1, keepdims=True)
        mil = jnp.where(lane == r, m, mil)
        first = jnp.min(jnp.where(work == m, idx, w), axis=1, keepdims=True)
        work = jnp.where(idx == first, -jnp.inf, work)
    work = jnp.where(kind >= R, a, jnp.inf)
    for r in range(R):
        m = jnp.min(work, axis=1, keepdims=True)
        mil = jnp.where(lane == R + r, m, mil)
        first = jnp.min(jnp.where(work == m, idx, w), axis=1, keepdims=True)
        work = jnp.where(idx == first, jnp.inf, work)

    mil = mil + b0_ref[0, 0]
    x = jax.nn.sigmoid(
        jnp.dot(mil, w1_ref[...], preferred_element_type=jnp.float32)
        + b1_ref[...])
    x = jax.nn.sigmoid(
        jnp.dot(x, w2_ref[...], preferred_element_type=jnp.float32)
        + b2_ref[...])
    out_ref[...] = jax.nn.sigmoid(
        jnp.dot(x, wo_ref[...], preferred_element_type=jnp.float32)
        + bo_ref[...])


def _head(cand, conv_b, fc1_w, fc1_b, fc2_w, fc2_b, fco_w, fco_b):
    n1, n2 = fc1_w.shape[0], fc2_w.shape[0]
    w1 = jnp.zeros((16, n1), jnp.float32).at[:2 * R].set(fc1_w.T)
    return pl.pallas_call(
        _head_body,
        out_shape=jax.ShapeDtypeStruct((B, 1), jnp.float32),
    )(cand, conv_b.reshape(1, 1), w1, fc1_b.reshape(1, n1), fc2_w.T,
      fc2_b.reshape(1, n2), fco_w.T, fco_b.reshape(1, 1))


def kernel(in_features, conv_w, conv_b, fc1_w, fc1_b, fc2_w, fc2_b, fco_w,
           fco_b):
    agg32 = _project(in_features, conv_w).reshape(NW, QLEN)
    cand = _select(agg32).reshape(B, NQ * CAND)
    out = _head(cand, conv_b, fc1_w, fc1_b, fc2_w, fc2_b, fco_w, fco_b)
    return out.reshape(B, 1, 1)


# speedup vs baseline: 1.0111x; 1.0111x over previous
import functools

import jax
import jax.numpy as jnp
from jax import lax
from jax.experimental import pallas as pl
from jax.experimental.pallas import tpu as pltpu
from jax.experimental.pallas import tpu_sc as plsc

B, C, N, R = 8, 2048, 8192, 5
CHUNK_N = 1024

NC = 1
NW = 16 * NC
NQ = NW // B
QLEN = B * N // NW
CAND = 2 * R * 16


def _proj_body(x_ref, w_ref, out_ref):
    x = x_ref[0]
    w = w_ref[...]
    out_ref[0] = jnp.sum(x * w, axis=0, keepdims=True)


def _project(in_features, conv_w):
    w_col = conv_w.reshape(C, 1)
    return pl.pallas_call(
        _proj_body,
        grid=(B, N // CHUNK_N),
        in_specs=[
            pl.BlockSpec((1, C, CHUNK_N), lambda b, n: (b, 0, n)),
            pl.BlockSpec((C, 1), lambda b, n: (0, 0)),
        ],
        out_specs=pl.BlockSpec((1, 1, CHUNK_N), lambda b, n: (b, 0, n)),
        out_shape=jax.ShapeDtypeStruct((B, 1, N), jnp.float32),
        compiler_params=pltpu.CompilerParams(
            dimension_semantics=("parallel", "parallel"),
        ),
    )(in_features, w_col)


def _select_body(agg_ref, out_ref, row_v, res_v):
    wid = lax.axis_index("s") * NC + lax.axis_index("c")
    pltpu.sync_copy(agg_ref.at[wid], row_v)
    neg = jnp.full((16,), -jnp.inf, jnp.float32)
    pos = jnp.full((16,), jnp.inf, jnp.float32)

    def body(i, carry):
        ts, us = list(carry[:R]), list(carry[R:])
        v = row_v[pl.ds(i * 16, 16)]
        w = v
        for k in range(R):
            hi = jnp.maximum(ts[k], w)
            w = jnp.minimum(ts[k], w)
            ts[k] = hi
        w = v
        for k in range(R):
            lo = jnp.minimum(us[k], w)
            w = jnp.maximum(us[k], w)
            us[k] = lo
        return tuple(ts) + tuple(us)

    carry = lax.fori_loop(0, QLEN // 16, body, (neg,) * R + (pos,) * R)
    for k in range(2 * R):
        res_v[pl.ds(16 * k, 16)] = carry[k]
    pltpu.sync_copy(res_v, out_ref.at[wid])


def _select(agg32):
    mesh = plsc.VectorSubcoreMesh(core_axis_name="c", subcore_axis_name="s",
                                  num_cores=NC)
    f = functools.partial(
        pl.kernel,
        out_type=jax.ShapeDtypeStruct((NW, CAND), jnp.float32),
        mesh=mesh,
        scratch_types=[
            pltpu.VMEM((QLEN,), jnp.float32),
            pltpu.VMEM((CAND,), jnp.float32),
        ],
    )(_select_body)
    return f(agg32)


def _head_body(cand_ref, b0_ref, w1_ref, b1_ref, w2_ref, b2_ref, wo_ref,
               bo_ref, out_ref):
    a = cand_ref[...]
    w = a.shape[1]
    idx = lax.broadcasted_iota(jnp.int32, (B, w), 1)
    kind = (idx % CAND) // 16
    lane = lax.broadcasted_iota(jnp.int32, (B, 16), 1)
    mil = jnp.zeros((B, 16), jnp.float32)

    work = jnp.where(kind < R, a, -jnp.inf)
    for r in range(R):
        m = jnp.max(work, axis=1, keepdims=True)
        mil = jnp.where(lane == r, m, mil)
        first = jnp.min(jnp.where(work == m, idx, w), axis=1, keepdims=True)
        work = jnp.where(idx == first, -jnp.inf, work)
    work = jnp.where(kind >= R, a, jnp.inf)
    for r in range(R):
        m = jnp.min(work, axis=1, keepdims=True)
        mil = jnp.where(lane == R + r, m, mil)
        first = jnp.min(jnp.where(work == m, idx, w), axis=1, keepdims=True)
        work = jnp.where(idx == first, jnp.inf, work)

    mil = mil + b0_ref[0, 0]
    x = jax.nn.sigmoid(
        jnp.dot(mil, w1_ref[...], preferred_element_type=jnp.float32)
        + b1_ref[...])
    x = jax.nn.sigmoid(
        jnp.dot(x, w2_ref[...], preferred_element_type=jnp.float32)
        + b2_ref[...])
    out_ref[...] = jax.nn.sigmoid(
        jnp.dot(x, wo_ref[...], preferred_element_type=jnp.float32)
        + bo_ref[...])


def _head(cand, conv_b, fc1_w, fc1_b, fc2_w, fc2_b, fco_w, fco_b):
    n1, n2 = fc1_w.shape[0], fc2_w.shape[0]
    w1 = jnp.zeros((16, n1), jnp.float32).at[:2 * R].set(fc1_w.T)
    return pl.pallas_call(
        _head_body,
        out_shape=jax.ShapeDtypeStruct((B, 1), jnp.float32),
    )(cand, conv_b.reshape(1, 1), w1, fc1_b.reshape(1, n1), fc2_w.T,
      fc2_b.reshape(1, n2), fco_w.T, fco_b.reshape(1, 1))


def kernel(in_features, conv_w, conv_b, fc1_w, fc1_b, fc2_w, fc2_b, fco_w,
           fco_b):
    agg16 = _project(in_features, conv_w).reshape(NW, QLEN)
    cand = _select(agg16).reshape(B, NQ * CAND)
    out = _head(cand, conv_b, fc1_w, fc1_b, fc2_w, fc2_b, fco_w, fco_b)
    return out.reshape(B, 1, 1)


# speedup vs baseline: 1.0224x; 1.0113x over previous
import functools

import jax
import jax.numpy as jnp
from jax import lax
from jax.experimental import pallas as pl
from jax.experimental.pallas import tpu as pltpu
from jax.experimental.pallas import tpu_sc as plsc

B, C, N, R = 8, 2048, 8192, 5
CHUNK_N = 1024

NC = 1
NW = 16 * NC
NQ = NW // B
QLEN = B * N // NW
CAND = 2 * R * 16


def _proj_body(x_ref, w_ref, out_ref):
    x = x_ref[0]
    w = w_ref[...]
    out_ref[0] = jnp.sum(x * w, axis=0, keepdims=True)


def _project(in_features, conv_w):
    w_col = conv_w.reshape(C, 1)
    return pl.pallas_call(
        _proj_body,
        grid=(B, N // CHUNK_N),
        in_specs=[
            pl.BlockSpec((1, C, CHUNK_N), lambda b, n: (b, 0, n)),
            pl.BlockSpec((C, 1), lambda b, n: (0, 0)),
        ],
        out_specs=pl.BlockSpec((1, 1, CHUNK_N), lambda b, n: (b, 0, n)),
        out_shape=jax.ShapeDtypeStruct((B, 1, N), jnp.float32),
        compiler_params=pltpu.CompilerParams(
            dimension_semantics=("parallel", "parallel"),
        ),
    )(in_features, w_col)


def _select_body(agg_ref, out_ref, row_v, res_v):
    wid = lax.axis_index("s") * NC + lax.axis_index("c")
    pltpu.sync_copy(
        agg_ref.at[wid // NQ, 0, pl.ds((wid % NQ) * QLEN, QLEN)], row_v)
    neg = jnp.full((16,), -jnp.inf, jnp.float32)
    pos = jnp.full((16,), jnp.inf, jnp.float32)

    def body(i, carry):
        ts, us = list(carry[:R]), list(carry[R:])
        v = row_v[pl.ds(i * 16, 16)]
        w = v
        for k in range(R):
            hi = jnp.maximum(ts[k], w)
            w = jnp.minimum(ts[k], w)
            ts[k] = hi
        w = v
        for k in range(R):
            lo = jnp.minimum(us[k], w)
            w = jnp.maximum(us[k], w)
            us[k] = lo
        return tuple(ts) + tuple(us)

    carry = lax.fori_loop(0, QLEN // 16, body, (neg,) * R + (pos,) * R)
    for k in range(2 * R):
        res_v[pl.ds(16 * k, 16)] = carry[k]
    pltpu.sync_copy(res_v, out_ref.at[wid])


def _select(agg32):
    mesh = plsc.VectorSubcoreMesh(core_axis_name="c", subcore_axis_name="s",
                                  num_cores=NC)
    f = functools.partial(
        pl.kernel,
        out_type=jax.ShapeDtypeStruct((NW, CAND), jnp.float32),
        mesh=mesh,
        scratch_types=[
            pltpu.VMEM((QLEN,), jnp.float32),
            pltpu.VMEM((CAND,), jnp.float32),
        ],
    )(_select_body)
    return f(agg32)


def _head_body(cand_ref, b0_ref, w1_ref, b1_ref, w2_ref, b2_ref, wo_ref,
               bo_ref, out_ref):
    a = cand_ref[...]
    w = a.shape[1]
    idx = lax.broadcasted_iota(jnp.int32, (B, w), 1)
    kind = (idx % CAND) // 16
    lane = lax.broadcasted_iota(jnp.int32, (B, 16), 1)
    mil = jnp.zeros((B, 16), jnp.float32)

    work = jnp.where(kind < R, a, -jnp.inf)
    for r in range(R):
        m = jnp.max(work, axis=1, keepdims=True)
        mil = jnp.where(lane == r, m, mil)
        first = jnp.min(jnp.where(work == m, idx, w), axis=1, keepdims=True)
        work = jnp.where(idx == first, -jnp.inf, work)
    work = jnp.where(kind >= R, a, jnp.inf)
    for r in range(R):
        m = jnp.min(work, axis=1, keepdims=True)
        mil = jnp.where(lane == R + r, m, mil)
        first = jnp.min(jnp.where(work == m, idx, w), axis=1, keepdims=True)
        work = jnp.where(idx == first, jnp.inf, work)

    mil = mil + b0_ref[0, 0]
    x = jax.nn.sigmoid(
        jnp.dot(mil, w1_ref[...], preferred_element_type=jnp.float32)
        + b1_ref[...])
    x = jax.nn.sigmoid(
        jnp.dot(x, w2_ref[...], preferred_element_type=jnp.float32)
        + b2_ref[...])
    out_ref[...] = jax.nn.sigmoid(
        jnp.dot(x, wo_ref[...], preferred_element_type=jnp.float32)
        + bo_ref[...])


def _head(cand, conv_b, fc1_w, fc1_b, fc2_w, fc2_b, fco_w, fco_b):
    n1, n2 = fc1_w.shape[0], fc2_w.shape[0]
    w1 = jnp.zeros((16, n1), jnp.float32).at[:2 * R].set(fc1_w.T)
    return pl.pallas_call(
        _head_body,
        out_shape=jax.ShapeDtypeStruct((B, 1), jnp.float32),
    )(cand, conv_b.reshape(1, 1), w1, fc1_b.reshape(1, n1), fc2_w.T,
      fc2_b.reshape(1, n2), fco_w.T, fco_b.reshape(1, 1))


def kernel(in_features, conv_w, conv_b, fc1_w, fc1_b, fc2_w, fc2_b, fco_w,
           fco_b):
    agg = _project(in_features, conv_w)
    cand = _select(agg).reshape(B, NQ * CAND)
    out = _head(cand, conv_b, fc1_w, fc1_b, fc2_w, fc2_b, fco_w, fco_b)
    return out.reshape(B, 1, 1)


# speedup vs baseline: 1.0290x; 1.0064x over previous
import functools

import jax
import jax.numpy as jnp
from jax import lax
from jax.experimental import pallas as pl
from jax.experimental.pallas import tpu as pltpu
from jax.experimental.pallas import tpu_sc as plsc

B, C, N, R = 8, 2048, 8192, 5
CHUNK_N = 1024

NC = 1
NW = 16 * NC
NQ = NW // B
QLEN = B * N // NW
CAND = 256


def _proj_body(x_ref, w_ref, out_ref):
    x = x_ref[0]
    w = w_ref[...]
    out_ref[0] = jnp.sum(x * w, axis=0, keepdims=True)


def _project(in_features, conv_w):
    w_col = conv_w.reshape(C, 1)
    return pl.pallas_call(
        _proj_body,
        grid=(B, N // CHUNK_N),
        in_specs=[
            pl.BlockSpec((1, C, CHUNK_N), lambda b, n: (b, 0, n)),
            pl.BlockSpec((C, 1), lambda b, n: (0, 0)),
        ],
        out_specs=pl.BlockSpec((1, 1, CHUNK_N), lambda b, n: (b, 0, n)),
        out_shape=jax.ShapeDtypeStruct((B, 1, N), jnp.float32),
        compiler_params=pltpu.CompilerParams(
            dimension_semantics=("parallel", "parallel"),
        ),
    )(in_features, w_col)


def _select_body(agg_ref, out_ref, row_v, res_v):
    wid = lax.axis_index("s") * NC + lax.axis_index("c")
    pltpu.sync_copy(
        agg_ref.at[wid // NQ, 0, pl.ds((wid % NQ) * QLEN, QLEN)], row_v)
    neg = jnp.full((16,), -jnp.inf, jnp.float32)
    pos = jnp.full((16,), jnp.inf, jnp.float32)

    def body(i, carry):
        ts, us = list(carry[:R]), list(carry[R:])
        v = row_v[pl.ds(i * 16, 16)]
        w = v
        for k in range(R):
            hi = jnp.maximum(ts[k], w)
            w = jnp.minimum(ts[k], w)
            ts[k] = hi
        w = v
        for k in range(R):
            lo = jnp.minimum(us[k], w)
            w = jnp.maximum(us[k], w)
            us[k] = lo
        return tuple(ts) + tuple(us)

    carry = lax.fori_loop(0, QLEN // 16, body, (neg,) * R + (pos,) * R)
    zero = jnp.zeros((16,), jnp.float32)
    for k in range(CAND // 16):
        res_v[pl.ds(16 * k, 16)] = carry[k] if k < 2 * R else zero
    pltpu.sync_copy(res_v,
                    out_ref.at[wid // NQ, 0, pl.ds((wid % NQ) * CAND, CAND)])


def _select(agg32):
    mesh = plsc.VectorSubcoreMesh(core_axis_name="c", subcore_axis_name="s",
                                  num_cores=NC)
    f = functools.partial(
        pl.kernel,
        out_type=jax.ShapeDtypeStruct((B, 1, NQ * CAND), jnp.float32),
        mesh=mesh,
        scratch_types=[
            pltpu.VMEM((QLEN,), jnp.float32),
            pltpu.VMEM((CAND,), jnp.float32),
        ],
    )(_select_body)
    return f(agg32)


def _head_body(cand_ref, b0_ref, w1_ref, b1_ref, w2_ref, b2_ref, wo_ref,
               bo_ref, out_ref):
    a = cand_ref[:, 0]
    w = a.shape[1]
    idx = lax.broadcasted_iota(jnp.int32, (B, w), 1)
    kind = (idx % CAND) // 16
    lane = lax.broadcasted_iota(jnp.int32, (B, 16), 1)
    mil = jnp.zeros((B, 16), jnp.float32)

    work = jnp.where(kind < R, a, -jnp.inf)
    for r in range(R):
        m = jnp.max(work, axis=1, keepdims=True)
        mil = jnp.where(lane == r, m, mil)
        first = jnp.min(jnp.where(work == m, idx, w), axis=1, keepdims=True)
        work = jnp.where(idx == first, -jnp.inf, work)
    work = jnp.where((kind >= R) & (kind < 2 * R), a, jnp.inf)
    for r in range(R):
        m = jnp.min(work, axis=1, keepdims=True)
        mil = jnp.where(lane == R + r, m, mil)
        first = jnp.min(jnp.where(work == m, idx, w), axis=1, keepdims=True)
        work = jnp.where(idx == first, jnp.inf, work)

    mil = mil + b0_ref[0, 0]
    x = jax.nn.sigmoid(
        jnp.dot(mil, w1_ref[...], preferred_element_type=jnp.float32)
        + b1_ref[...])
    x = jax.nn.sigmoid(
        jnp.dot(x, w2_ref[...], preferred_element_type=jnp.float32)
        + b2_ref[...])
    out_ref[...] = jax.nn.sigmoid(
        jnp.dot(x, wo_ref[...], preferred_element_type=jnp.float32)
        + bo_ref[...])


def _head(cand, conv_b, fc1_w, fc1_b, fc2_w, fc2_b, fco_w, fco_b):
    n1, n2 = fc1_w.shape[0], fc2_w.shape[0]
    w1 = jnp.zeros((16, n1), jnp.float32).at[:2 * R].set(fc1_w.T)
    return pl.pallas_call(
        _head_body,
        out_shape=jax.ShapeDtypeStruct((B, 1), jnp.float32),
    )(cand, conv_b.reshape(1, 1), w1, fc1_b.reshape(1, n1), fc2_w.T,
      fc2_b.reshape(1, n2), fco_w.T, fco_b.reshape(1, 1))


def kernel(in_features, conv_w, conv_b, fc1_w, fc1_b, fc2_w, fc2_b, fco_w,
           fco_b):
    agg = _project(in_features, conv_w)
    cand = _select(agg)
    out = _head(cand, conv_b, fc1_w, fc1_b, fc2_w, fc2_b, fco_w, fco_b)
    return out.reshape(B, 1, 1)
